# Initial kernel scaffold; baseline (speedup 1.0000x reference)
import jax
import jax.numpy as jnp
from jax.experimental import pallas as pl


def kernel(x, W3, W4, W5):
    raise NotImplementedError("write your pallas kernel here")


# speedup vs baseline: 7.7049x; 7.7049x over previous
import functools

import jax
import jax.numpy as jnp
from jax import lax
from jax.experimental import pallas as pl
from jax.experimental.pallas import tpu as pltpu, tpu_sc as plsc

HTS = 50000
D = 128
BATCH = 4
T = 8192
NW = 32
TOK_PER_W = (BATCH * T) // NW
CHUNK = 128
NCHUNK = TOK_PER_W // CHUNK
XROW = T + 8

P2 = 66049
P3 = 16974593
P4 = 67503105
WRAP_FIX = 17296


def _umod_hts(h):
    r = lax.rem(h, jnp.int32(HTS))
    r = r + jnp.where(r < 0, jnp.int32(HTS), jnp.int32(0))
    r = r + jnp.where(h < 0, jnp.int32(WRAP_FIX), jnp.int32(0))
    r = r - jnp.where(r >= jnp.int32(HTS), jnp.int32(HTS), jnp.int32(0))
    return r


def _body(xf, w3, w4, w5, out, xbuf, i3, i4, i5, b3, b4, b5, sem):
    i32 = jnp.int32
    c = lax.axis_index("c")
    s = lax.axis_index("s")
    wid = s * i32(2) + c
    row = wid // i32(8)
    seg = wid - row * i32(8)
    xoff = row * i32(XROW) + seg * i32(TOK_PER_W)
    outrow = wid * i32(TOK_PER_W)

    pltpu.sync_copy(xf.at[pl.ds(xoff, TOK_PER_W + 8)], xbuf)

    def chunk_body(k, carry):
        off = k * i32(CHUNK)
        for g in range(CHUNK // 16):
            o = off + i32(g * 16)
            v0 = xbuf[pl.ds(o + i32(4), 16)]
            v1 = xbuf[pl.ds(o + i32(3), 16)]
            v2 = xbuf[pl.ds(o + i32(2), 16)]
            v3 = xbuf[pl.ds(o + i32(1), 16)]
            v4 = xbuf[pl.ds(o, 16)]
            h3 = v2 * jnp.int32(P2) + v1 * jnp.int32(257) + v0
            h4 = h3 + v3 * jnp.int32(P3)
            h5 = h4 + v4 * jnp.int32(P4)
            sl = pl.ds(g * 16, 16)
            i3[sl] = _umod_hts(h3)
            i4[sl] = _umod_hts(h4)
            i5[sl] = _umod_hts(h5)

        d3 = pltpu.async_copy(w3.at[i3], b3, sem)
        d4 = pltpu.async_copy(w4.at[i4], b4, sem)
        d5 = pltpu.async_copy(w5.at[i5], b5, sem)
        d3.wait()
        d4.wait()
        d5.wait()

        @pl.when(jnp.logical_and(seg == i32(0), k == 0))
        def _zero_edge():
            z = jnp.zeros((16,), jnp.float32)
            for col in range(D // 16):
                csl = pl.ds(col * 16, 16)
                for r in range(2):
                    b3[r, csl] = z
                for r in range(3):
                    b4[r, csl] = z
                for r in range(4):
                    b5[r, csl] = z

        third = jnp.float32(1.0 / 3.0)

        def acc_body(tt, carry2):
            for col in range(D // 16):
                csl = pl.ds(col * 16, 16)
                b3[tt, csl] = (b3[tt, csl] + b4[tt, csl] + b5[tt, csl]) * third
            return carry2

        lax.fori_loop(jnp.int32(0), jnp.int32(CHUNK), acc_body, jnp.int32(0))

        pltpu.sync_copy(b3, out.at[pl.ds(outrow + off, CHUNK)])
        return carry

    lax.fori_loop(jnp.int32(0), jnp.int32(NCHUNK), chunk_body, jnp.int32(0))


@functools.partial(jax.jit, static_argnames=())
def _sc_embed(xflat, w3, w4, w5):
    mesh = plsc.VectorSubcoreMesh(core_axis_name="c", subcore_axis_name="s")
    f = pl.kernel(
        _body,
        out_type=jax.ShapeDtypeStruct((BATCH * T, D), jnp.float32),
        mesh=mesh,
        scratch_types=[
            pltpu.VMEM((TOK_PER_W + 8,), jnp.int32),
            pltpu.VMEM((CHUNK,), jnp.int32),
            pltpu.VMEM((CHUNK,), jnp.int32),
            pltpu.VMEM((CHUNK,), jnp.int32),
            pltpu.VMEM((CHUNK, D), jnp.float32),
            pltpu.VMEM((CHUNK, D), jnp.float32),
            pltpu.VMEM((CHUNK, D), jnp.float32),
            pltpu.SemaphoreType.DMA,
        ],
    )
    return f(xflat, w3, w4, w5)


def kernel(x, W3, W4, W5):
    x32 = x.astype(jnp.int32)
    xpad = jnp.pad(x32, ((0, 0), (4, 4)))
    xflat = xpad.reshape(-1)
    out = _sc_embed(xflat, W3, W4, W5)
    return out.reshape(BATCH, T, D)


# speedup vs baseline: 11.7793x; 1.5288x over previous
import jax
import jax.numpy as jnp
import numpy as np
from jax import lax
from jax.experimental import pallas as pl
from jax.experimental.pallas import tpu as pltpu, tpu_sc as plsc

HTS = 50000
D = 128
BATCH = 4
T = 8192
NW = 32
TOK_PER_W = (BATCH * T) // NW
CHUNK = 128
NCHUNK = TOK_PER_W // CHUNK
XROW = T + 8

P2 = 66049
P3 = 16974593
P4 = 67503105
WRAP_FIX = 17296


def _umod_hts(h):
    r = lax.rem(h, jnp.int32(HTS))
    r = r + jnp.where(r < 0, jnp.int32(HTS), jnp.int32(0))
    r = r + jnp.where(h < 0, jnp.int32(WRAP_FIX), jnp.int32(0))
    r = r - jnp.where(r >= jnp.int32(HTS), jnp.int32(HTS), jnp.int32(0))
    return r


def _body(xf, w3, w4, w5, out,
          xbuf,
          i3a, i4a, i5a, i3b, i4b, i5b,
          b3a, b4a, b5a, b3b, b4b, b5b,
          g0, g1, ws0, ws1):
    i32 = jnp.int32
    c = lax.axis_index("c")
    s = lax.axis_index("s")
    wid = s * i32(2) + c
    row = wid // i32(8)
    seg = wid - row * i32(8)
    xoff = row * i32(XROW) + seg * i32(TOK_PER_W)
    outrow = wid * i32(TOK_PER_W)

    idx_sets = [(i3a, i4a, i5a), (i3b, i4b, i5b)]
    buf_sets = [(b3a, b4a, b5a), (b3b, b4b, b5b)]
    gsems = [g0, g1]
    wsems = [ws0, ws1]
    third = jnp.float32(1.0 / 3.0)

    pltpu.sync_copy(xf.at[pl.ds(xoff, TOK_PER_W + 8)], xbuf)

    def compute_idx(obase, iset):
        i3, i4, i5 = iset

        @pl.loop(i32(0), i32(CHUNK), step=i32(16))
        def _grp(gg):
            o = obase + gg
            v0 = xbuf[pl.ds(o + i32(4), 16)]
            v1 = xbuf[pl.ds(o + i32(3), 16)]
            v2 = xbuf[pl.ds(o + i32(2), 16)]
            v3 = xbuf[pl.ds(o + i32(1), 16)]
            v4 = xbuf[pl.ds(o, 16)]
            h3 = v2 * jnp.int32(P2) + v1 * jnp.int32(257) + v0
            h4 = h3 + v3 * jnp.int32(P3)
            h5 = h4 + v4 * jnp.int32(P4)
            sl = pl.ds(gg, 16)
            i3[sl] = _umod_hts(h3)
            i4[sl] = _umod_hts(h4)
            i5[sl] = _umod_hts(h5)

    def fire_gathers(b):
        i3, i4, i5 = idx_sets[b]
        b3, b4, b5 = buf_sets[b]
        pltpu.async_copy(w3.at[i3], b3, gsems[b])
        pltpu.async_copy(w4.at[i4], b4, gsems[b])
        pltpu.async_copy(w5.at[i5], b5, gsems[b])

    def drain_gathers(b):
        i3, i4, i5 = idx_sets[b]
        b3, b4, b5 = buf_sets[b]
        pltpu.make_async_copy(w3.at[i3], b3, gsems[b]).wait()
        pltpu.make_async_copy(w4.at[i4], b4, gsems[b]).wait()
        pltpu.make_async_copy(w5.at[i5], b5, gsems[b]).wait()

    def fire_wb(b, obase):
        b3 = buf_sets[b][0]
        pltpu.async_copy(b3, out.at[pl.ds(outrow + obase, CHUNK)], wsems[b])

    def drain_wb(b, obase):
        b3 = buf_sets[b][0]
        pltpu.make_async_copy(
            b3, out.at[pl.ds(outrow + obase, CHUNK)], wsems[b]).wait()

    def zero_edge(b, kk_dyn):
        b3, b4, b5 = buf_sets[b]

        @pl.when((seg == i32(0)) & (kk_dyn == i32(0)))
        def _zero():
            z = jnp.zeros((16,), jnp.float32)
            for col in range(D // 16):
                csl = pl.ds(col * 16, 16)
                for r in range(2):
                    b3[r, csl] = z
                for r in range(3):
                    b4[r, csl] = z
                for r in range(4):
                    b5[r, csl] = z

    def accumulate(b):
        b3, b4, b5 = buf_sets[b]

        @plsc.parallel_loop(i32(0), i32(CHUNK), i32(1), unroll=4)
        def _acc(tt):
            for col in range(D // 16):
                csl = pl.ds(col * 16, 16)
                b3[tt, csl] = (b3[tt, csl] + b4[tt, csl] + b5[tt, csl]) * third

    for b in range(2):
        compute_idx(i32(b * CHUNK), idx_sets[b])
        fire_gathers(b)

    @pl.loop(i32(0), i32(NCHUNK), step=i32(2))
    def _main(k):
        for b in range(2):
            kk = k + i32(b)
            obase = kk * i32(CHUNK)
            drain_gathers(b)
            zero_edge(b, kk)
            accumulate(b)
            fire_wb(b, obase)

            @pl.when(kk < i32(NCHUNK - 2))
            def _next():
                compute_idx(obase + i32(2 * CHUNK), idx_sets[b])
                drain_wb(b, obase)
                fire_gathers(b)

    for b in range(2):
        drain_wb(b, i32((NCHUNK - 2 + b) * CHUNK))


@jax.jit
def _sc_embed(xflat, w3, w4, w5):
    mesh = plsc.VectorSubcoreMesh(core_axis_name="c", subcore_axis_name="s")
    f = pl.kernel(
        _body,
        out_type=jax.ShapeDtypeStruct((BATCH * T, D), jnp.float32),
        mesh=mesh,
        scratch_types=[
            pltpu.VMEM((TOK_PER_W + 8,), jnp.int32),
            pltpu.VMEM((CHUNK,), jnp.int32),
            pltpu.VMEM((CHUNK,), jnp.int32),
            pltpu.VMEM((CHUNK,), jnp.int32),
            pltpu.VMEM((CHUNK,), jnp.int32),
            pltpu.VMEM((CHUNK,), jnp.int32),
            pltpu.VMEM((CHUNK,), jnp.int32),
            pltpu.VMEM((CHUNK, D), jnp.float32),
            pltpu.VMEM((CHUNK, D), jnp.float32),
            pltpu.VMEM((CHUNK, D), jnp.float32),
            pltpu.VMEM((CHUNK, D), jnp.float32),
            pltpu.VMEM((CHUNK, D), jnp.float32),
            pltpu.VMEM((CHUNK, D), jnp.float32),
            pltpu.SemaphoreType.DMA,
            pltpu.SemaphoreType.DMA,
            pltpu.SemaphoreType.DMA,
            pltpu.SemaphoreType.DMA,
        ],
    )
    return f(xflat, w3, w4, w5)


def kernel(x, W3, W4, W5):
    x32 = x.astype(jnp.int32)
    xpad = jnp.pad(x32, ((0, 0), (4, 4)))
    xflat = xpad.reshape(-1)
    out = _sc_embed(xflat, W3, W4, W5)
    return out.reshape(BATCH, T, D)


# speedup vs baseline: 12.3897x; 1.0518x over previous
import jax
import jax.numpy as jnp
import numpy as np
from jax import lax
from jax.experimental import pallas as pl
from jax.experimental.pallas import tpu as pltpu, tpu_sc as plsc

HTS = 50000
D = 128
BATCH = 4
T = 8192
NW = 32
TOK_PER_W = (BATCH * T) // NW
CHUNK = 256
NCHUNK = TOK_PER_W // CHUNK
XROW = T + 8

P2 = 66049
P3 = 16974593
P4 = 67503105
WRAP_FIX = 17296


def _umod_hts(h):
    r = lax.rem(h, jnp.int32(HTS))
    r = r + jnp.where(r < 0, jnp.int32(HTS), jnp.int32(0))
    r = r + jnp.where(h < 0, jnp.int32(WRAP_FIX), jnp.int32(0))
    r = r - jnp.where(r >= jnp.int32(HTS), jnp.int32(HTS), jnp.int32(0))
    return r


def _body(xf, w3, w4, w5, out,
          xbuf,
          i3a, i4a, i5a, i3b, i4b, i5b,
          acc_a, acc_b,
          e3, e4, obuf,
          g0, g1, ws0, ws1, es):
    i32 = jnp.int32
    c = lax.axis_index("c")
    s = lax.axis_index("s")
    wid = s * i32(2) + c
    row = wid // i32(8)
    seg = wid - row * i32(8)
    xoff = row * i32(XROW) + seg * i32(TOK_PER_W)
    outrow = wid * i32(TOK_PER_W)

    idx_sets = [(i3a, i4a, i5a), (i3b, i4b, i5b)]
    accs = [acc_a, acc_b]
    gsems = [g0, g1]
    wsems = [ws0, ws1]
    third = jnp.float32(1.0 / 3.0)

    pltpu.sync_copy(xf.at[pl.ds(xoff, TOK_PER_W + 8)], xbuf)

    def compute_idx(obase, iset):
        i3, i4, i5 = iset

        @pl.loop(i32(0), i32(CHUNK), step=i32(16))
        def _grp(gg):
            o = obase + gg
            v0 = xbuf[pl.ds(o + i32(4), 16)]
            v1 = xbuf[pl.ds(o + i32(3), 16)]
            v2 = xbuf[pl.ds(o + i32(2), 16)]
            v3 = xbuf[pl.ds(o + i32(1), 16)]
            v4 = xbuf[pl.ds(o, 16)]
            h3 = v2 * jnp.int32(P2) + v1 * jnp.int32(257) + v0
            h4 = h3 + v3 * jnp.int32(P3)
            h5 = h4 + v4 * jnp.int32(P4)
            sl = pl.ds(gg, 16)
            i3[sl] = _umod_hts(h3)
            i4[sl] = _umod_hts(h4)
            i5[sl] = _umod_hts(h5)

    def zero_acc(b):
        acc = accs[b]
        z = jnp.zeros((16,), jnp.float32)

        @plsc.parallel_loop(i32(0), i32(CHUNK), i32(1), unroll=4)
        def _z(tt):
            for col in range(D // 16):
                acc[tt, pl.ds(col * 16, 16)] = z

    def fire_gathers(b):
        i3, i4, i5 = idx_sets[b]
        acc = accs[b]
        pltpu.async_copy(w3.at[i3], acc, gsems[b], add=True)
        pltpu.async_copy(w4.at[i4], acc, gsems[b], add=True)
        pltpu.async_copy(w5.at[i5], acc, gsems[b], add=True)

    def drain_gathers(b):
        i3, i4, i5 = idx_sets[b]
        acc = accs[b]
        pltpu.make_async_copy(w3.at[i3], acc, gsems[b]).wait()
        pltpu.make_async_copy(w4.at[i4], acc, gsems[b]).wait()
        pltpu.make_async_copy(w5.at[i5], acc, gsems[b]).wait()

    def scale(b):
        acc = accs[b]

        @plsc.parallel_loop(i32(0), i32(CHUNK), i32(1), unroll=4)
        def _sc(tt):
            for col in range(D // 16):
                csl = pl.ds(col * 16, 16)
                acc[tt, csl] = acc[tt, csl] * third

    def fire_wb(b, obase):
        pltpu.async_copy(accs[b], out.at[pl.ds(outrow + obase, CHUNK)],
                         wsems[b])

    def drain_wb(b, obase):
        pltpu.make_async_copy(
            accs[b], out.at[pl.ds(outrow + obase, CHUNK)], wsems[b]).wait()

    for b in range(2):
        zero_acc(b)
        compute_idx(i32(b * CHUNK), idx_sets[b])
        fire_gathers(b)

    pltpu.async_copy(w3.at[i3a.at[pl.ds(0, 16)]], e3, es)
    pltpu.async_copy(w4.at[i4a.at[pl.ds(0, 16)]], e4, es)

    @pl.loop(i32(0), i32(NCHUNK), step=i32(2))
    def _main(k):
        for b in range(2):
            kk = k + i32(b)
            obase = kk * i32(CHUNK)
            drain_gathers(b)
            scale(b)
            fire_wb(b, obase)

            @pl.when(kk < i32(NCHUNK - 2))
            def _next():
                compute_idx(obase + i32(2 * CHUNK), idx_sets[b])
                drain_wb(b, obase)
                zero_acc(b)
                fire_gathers(b)

    for b in range(2):
        drain_wb(b, i32((NCHUNK - 2 + b) * CHUNK))

    pltpu.make_async_copy(w3.at[i3a.at[pl.ds(0, 16)]], e3, es).wait()
    pltpu.make_async_copy(w4.at[i4a.at[pl.ds(0, 16)]], e4, es).wait()

    @pl.when(seg == i32(0))
    def _edge():
        z = jnp.zeros((16,), jnp.float32)
        for col in range(D // 16):
            csl = pl.ds(col * 16, 16)
            obuf[0, csl] = z
            obuf[1, csl] = z
            obuf[2, csl] = e3[2, csl] * third
            obuf[3, csl] = (e3[3, csl] + e4[3, csl]) * third
        pltpu.sync_copy(obuf, out.at[pl.ds(outrow, 4)])


@jax.jit
def _sc_embed(xflat, w3, w4, w5):
    mesh = plsc.VectorSubcoreMesh(core_axis_name="c", subcore_axis_name="s")
    f = pl.kernel(
        _body,
        out_type=jax.ShapeDtypeStruct((BATCH * T, D), jnp.float32),
        mesh=mesh,
        scratch_types=[
            pltpu.VMEM((TOK_PER_W + 8,), jnp.int32),
            pltpu.VMEM((CHUNK,), jnp.int32),
            pltpu.VMEM((CHUNK,), jnp.int32),
            pltpu.VMEM((CHUNK,), jnp.int32),
            pltpu.VMEM((CHUNK,), jnp.int32),
            pltpu.VMEM((CHUNK,), jnp.int32),
            pltpu.VMEM((CHUNK,), jnp.int32),
            pltpu.VMEM((CHUNK, D), jnp.float32),
            pltpu.VMEM((CHUNK, D), jnp.float32),
            pltpu.VMEM((16, D), jnp.float32),
            pltpu.VMEM((16, D), jnp.float32),
            pltpu.VMEM((4, D), jnp.float32),
            pltpu.SemaphoreType.DMA,
            pltpu.SemaphoreType.DMA,
            pltpu.SemaphoreType.DMA,
            pltpu.SemaphoreType.DMA,
            pltpu.SemaphoreType.DMA,
        ],
    )
    return f(xflat, w3, w4, w5)


def kernel(x, W3, W4, W5):
    x32 = x.astype(jnp.int32)
    xpad = jnp.pad(x32, ((0, 0), (4, 4)))
    xflat = xpad.reshape(-1)
    out = _sc_embed(xflat, W3, W4, W5)
    return out.reshape(BATCH, T, D)


# speedup vs baseline: 12.5010x; 1.0090x over previous
import jax
import jax.numpy as jnp
import numpy as np
from jax import lax
from jax.experimental import pallas as pl
from jax.experimental.pallas import tpu as pltpu, tpu_sc as plsc

HTS = 50000
D = 128
BATCH = 4
T = 8192
NW = 32
TOK_PER_W = (BATCH * T) // NW
CHUNK = 256
NCHUNK = TOK_PER_W // CHUNK
XROW = T + 8

P2 = 66049
P3 = 16974593
P4 = 67503105
WRAP_FIX = 17296


def _umod_hts(h):
    r = lax.rem(h, jnp.int32(HTS))
    r = r + jnp.where(r < 0, jnp.int32(HTS), jnp.int32(0))
    r = r + jnp.where(h < 0, jnp.int32(WRAP_FIX), jnp.int32(0))
    r = r - jnp.where(r >= jnp.int32(HTS), jnp.int32(HTS), jnp.int32(0))
    return r


def _body(xf, w3, w4, w5, out,
          xbuf,
          i3a, i4a, i5a, i3b, i4b, i5b,
          acc_a, acc_b,
          e3, e4, obuf,
          g0, g1, ws0, ws1, es):
    i32 = jnp.int32
    c = lax.axis_index("c")
    s = lax.axis_index("s")
    wid = s * i32(2) + c
    row = wid // i32(8)
    seg = wid - row * i32(8)
    xoff = row * i32(XROW) + seg * i32(TOK_PER_W)
    outrow = wid * i32(TOK_PER_W)

    idx_sets = [(i3a, i4a, i5a), (i3b, i4b, i5b)]
    accs = [acc_a, acc_b]
    gsems = [g0, g1]
    wsems = [ws0, ws1]
    third = jnp.float32(1.0 / 3.0)

    pltpu.sync_copy(xf.at[pl.ds(xoff, TOK_PER_W + 8)], xbuf)

    def compute_idx(obase, iset):
        i3, i4, i5 = iset

        @pl.loop(i32(0), i32(CHUNK), step=i32(16))
        def _grp(gg):
            o = obase + gg
            v0 = xbuf[pl.ds(o + i32(4), 16)]
            v1 = xbuf[pl.ds(o + i32(3), 16)]
            v2 = xbuf[pl.ds(o + i32(2), 16)]
            v3 = xbuf[pl.ds(o + i32(1), 16)]
            v4 = xbuf[pl.ds(o, 16)]
            h3 = v2 * jnp.int32(P2) + v1 * jnp.int32(257) + v0
            h4 = h3 + v3 * jnp.int32(P3)
            h5 = h4 + v4 * jnp.int32(P4)
            sl = pl.ds(gg, 16)
            i3[sl] = _umod_hts(h3)
            i4[sl] = _umod_hts(h4)
            i5[sl] = _umod_hts(h5)

    def zero_acc(b):
        acc = accs[b]
        z = jnp.zeros((16,), jnp.float32)

        @plsc.parallel_loop(i32(0), i32(CHUNK), i32(1))
        def _z(tt):
            for col in range(D // 16):
                acc[tt, pl.ds(col * 16, 16)] = z

    def fire_gathers(b):
        i3, i4, i5 = idx_sets[b]
        acc = accs[b]
        pltpu.async_copy(w3.at[i3], acc, gsems[b], add=True)
        pltpu.async_copy(w4.at[i4], acc, gsems[b], add=True)
        pltpu.async_copy(w5.at[i5], acc, gsems[b], add=True)

    def drain_gathers(b):
        i3, i4, i5 = idx_sets[b]
        acc = accs[b]
        pltpu.make_async_copy(w3.at[i3], acc, gsems[b]).wait()
        pltpu.make_async_copy(w4.at[i4], acc, gsems[b]).wait()
        pltpu.make_async_copy(w5.at[i5], acc, gsems[b]).wait()

    def scale(b):
        acc = accs[b]

        @plsc.parallel_loop(i32(0), i32(CHUNK), i32(1), unroll=2)
        def _sc(tt):
            for col in range(D // 16):
                csl = pl.ds(col * 16, 16)
                acc[tt, csl] = acc[tt, csl] * third

    def fire_wb(b, obase):
        pltpu.async_copy(accs[b], out.at[pl.ds(outrow + obase, CHUNK)],
                         wsems[b])

    def drain_wb(b, obase):
        pltpu.make_async_copy(
            accs[b], out.at[pl.ds(outrow + obase, CHUNK)], wsems[b]).wait()

    for b in range(2):
        zero_acc(b)
        compute_idx(i32(b * CHUNK), idx_sets[b])
        fire_gathers(b)

    pltpu.async_copy(w3.at[i3a.at[pl.ds(0, 16)]], e3, es)
    pltpu.async_copy(w4.at[i4a.at[pl.ds(0, 16)]], e4, es)

    @pl.loop(i32(0), i32(NCHUNK), step=i32(2))
    def _main(k):
        for b in range(2):
            kk = k + i32(b)
            obase = kk * i32(CHUNK)
            drain_gathers(b)
            scale(b)
            fire_wb(b, obase)

            @pl.when(kk < i32(NCHUNK - 2))
            def _next():
                compute_idx(obase + i32(2 * CHUNK), idx_sets[b])
                drain_wb(b, obase)
                zero_acc(b)
                fire_gathers(b)

    for b in range(2):
        drain_wb(b, i32((NCHUNK - 2 + b) * CHUNK))

    pltpu.make_async_copy(w3.at[i3a.at[pl.ds(0, 16)]], e3, es).wait()
    pltpu.make_async_copy(w4.at[i4a.at[pl.ds(0, 16)]], e4, es).wait()

    @pl.when(seg == i32(0))
    def _edge():
        z = jnp.zeros((16,), jnp.float32)
        for col in range(D // 16):
            csl = pl.ds(col * 16, 16)
            obuf[0, csl] = z
            obuf[1, csl] = z
            obuf[2, csl] = e3[2, csl] * third
            obuf[3, csl] = (e3[3, csl] + e4[3, csl]) * third
        pltpu.sync_copy(obuf, out.at[pl.ds(outrow, 4)])


@jax.jit
def _sc_embed(xflat, w3, w4, w5):
    mesh = plsc.VectorSubcoreMesh(core_axis_name="c", subcore_axis_name="s")
    f = pl.kernel(
        _body,
        out_type=jax.ShapeDtypeStruct((BATCH * T, D), jnp.float32),
        mesh=mesh,
        scratch_types=[
            pltpu.VMEM((TOK_PER_W + 8,), jnp.int32),
            pltpu.VMEM((CHUNK,), jnp.int32),
            pltpu.VMEM((CHUNK,), jnp.int32),
            pltpu.VMEM((CHUNK,), jnp.int32),
            pltpu.VMEM((CHUNK,), jnp.int32),
            pltpu.VMEM((CHUNK,), jnp.int32),
            pltpu.VMEM((CHUNK,), jnp.int32),
            pltpu.VMEM((CHUNK, D), jnp.float32),
            pltpu.VMEM((CHUNK, D), jnp.float32),
            pltpu.VMEM((16, D), jnp.float32),
            pltpu.VMEM((16, D), jnp.float32),
            pltpu.VMEM((4, D), jnp.float32),
            pltpu.SemaphoreType.DMA,
            pltpu.SemaphoreType.DMA,
            pltpu.SemaphoreType.DMA,
            pltpu.SemaphoreType.DMA,
            pltpu.SemaphoreType.DMA,
        ],
    )
    return f(xflat, w3, w4, w5)


def kernel(x, W3, W4, W5):
    x32 = x.astype(jnp.int32)
    xpad = jnp.pad(x32, ((0, 0), (4, 4)))
    xflat = xpad.reshape(-1)
    out = _sc_embed(xflat, W3, W4, W5)
    return out.reshape(BATCH, T, D)
